# baseline (speedup 1.0000x reference)
import functools

import jax
import jax.numpy as jnp
from jax import lax
from jax.experimental import pallas as pl
from jax.experimental.pallas import tpu as pltpu
from jax.experimental.pallas import tpu_sc as plsc

N = 50000
E = 800000
BR = 2000
NB = N // BR
CHUNK = 128
N_TILES = 16
N_CHUNKS = 392
EDGES_PER_TILE = N_CHUNKS * CHUNK
E_PAD = N_TILES * EDGES_PER_TILE
ROWS_PER_TILE = 3200
N_ACC = N_TILES * ROWS_PER_TILE
F32 = jnp.float32


def _sc_agg_body(use_ew, qmul, cstride, poff, table, src, dst, ew, out,
                 src_i, dst_i, ew_b, msg, acc, ssem, dsem, esem, gsem):
    c = lax.axis_index("c")
    s = lax.axis_index("s")
    r0 = s * ROWS_PER_TILE

    def _zrow(i, _):
        msg[0, i, pl.ds(0, 16)] = jnp.zeros((16,), F32)
        msg[0, i, pl.ds(16, 16)] = jnp.zeros((16,), F32)
        return 0
    lax.fori_loop(0, CHUNK, _zrow, 0)

    def _zacc(k, _):
        pltpu.sync_copy(msg.at[0], acc.at[pl.ds(r0 + k * CHUNK, CHUNK)])
        return 0
    lax.fori_loop(0, ROWS_PER_TILE // CHUNK, _zacc, 0)
    plsc.subcore_barrier()

    base = s * EDGES_PER_TILE
    coff = c * cstride + poff

    def start_idx(j, b):
        off = base + j * CHUNK
        pltpu.async_copy(src.at[pl.ds(off, CHUNK)], src_i.at[b], ssem.at[b])
        pltpu.async_copy(dst.at[pl.ds(off, CHUNK)], dst_i.at[b], dsem.at[b])
        if use_ew:
            pltpu.async_copy(ew.at[pl.ds(off, CHUNK)], ew_b.at[b],
                             esem.at[b])

    def gather_chunk(b):
        pltpu.make_async_copy(src.at[pl.ds(base, CHUNK)], src_i.at[b],
                              ssem.at[b]).wait()
        for f in range(CHUNK // 16):
            sl = pl.ds(f * 16, 16)
            if qmul == 1:
                src_i[b, sl] = src_i[b, sl] + coff
            else:
                src_i[b, sl] = src_i[b, sl] * qmul + coff
        pltpu.async_copy(table.at[src_i.at[b]], msg.at[b], gsem.at[b])

    def wait_gather(b):
        pltpu.make_async_copy(table.at[src_i.at[b]], msg.at[b],
                              gsem.at[b]).wait()

    def do_scatter(b):
        if use_ew:
            pltpu.make_async_copy(ew.at[pl.ds(base, CHUNK)], ew_b.at[b],
                                  esem.at[b]).wait()
            for g in range(CHUNK // 16):
                w16 = ew_b[b, pl.ds(g * 16, 16)]
                for el in range(16):
                    e = g * 16 + el
                    wv = jnp.full((16,), w16[el], F32)
                    msg[b, e, pl.ds(0, 16)] = msg[b, e, pl.ds(0, 16)] * wv
                    msg[b, e, pl.ds(16, 16)] = msg[b, e, pl.ds(16, 16)] * wv
        pltpu.make_async_copy(dst.at[pl.ds(base, CHUNK)], dst_i.at[b],
                              dsem.at[b]).wait()
        pltpu.sync_copy(msg.at[b], acc.at[dst_i.at[b]], add=True)

    start_idx(0, 0)
    start_idx(1, 1)
    gather_chunk(0)

    def _outer(t, _):
        j0 = 2 * t
        for b in range(2):
            wait_gather(b)
            gather_chunk(1 - b)
            do_scatter(b)
            start_idx(j0 + b + 2, b)
        return 0
    lax.fori_loop(0, (N_CHUNKS - 2) // 2, _outer, 0)
    wait_gather(0)
    gather_chunk(1)
    do_scatter(0)
    wait_gather(1)
    do_scatter(1)
    plsc.subcore_barrier()

    def _drain(k, _):
        r = r0 + k * CHUNK
        pltpu.sync_copy(acc.at[pl.ds(r, CHUNK)], msg.at[0])
        pltpu.sync_copy(msg.at[0], out.at[c, pl.ds(r, CHUNK)])
        return 0
    lax.fori_loop(0, ROWS_PER_TILE // CHUNK, _drain, 0)


@functools.cache
def _build_sc_agg(use_ew, qmul, cstride, poff, table_rows):
    mesh = plsc.VectorSubcoreMesh(core_axis_name="c", subcore_axis_name="s",
                                  num_cores=2, num_subcores=N_TILES)
    return pl.kernel(
        functools.partial(_sc_agg_body, use_ew, qmul, cstride, poff),
        out_type=jax.ShapeDtypeStruct((2, N_ACC, 32), F32),
        mesh=mesh,
        scratch_types=[
            pltpu.VMEM((2, CHUNK), jnp.int32),
            pltpu.VMEM((2, CHUNK), jnp.int32),
            pltpu.VMEM((2, CHUNK), F32),
            pltpu.VMEM((2, CHUNK, 32), F32),
            pltpu.VMEM_SHARED((N_ACC, 32), F32),
            pltpu.SemaphoreType.DMA((2,)),
            pltpu.SemaphoreType.DMA((2,)),
            pltpu.SemaphoreType.DMA((2,)),
            pltpu.SemaphoreType.DMA((2,)),
        ],
        compiler_params=pltpu.CompilerParams(use_tc_tiling_on_sc=False),
    )


BF16 = jnp.bfloat16


def _dot1(a, b):
    return jnp.dot(a.astype(BF16), b.astype(BF16), preferred_element_type=F32)


def _sum_stats_body(z, st_ref, i):
    @pl.when(i == 0)
    def _():
        st_ref[...] = jnp.zeros_like(st_ref)
    st_ref[...] += jnp.stack([jnp.sum(z, 0), jnp.sum(z * z, 0)])


def _bn(z, st, g, b):
    m = st[0] / N
    v = st[1] / N - m * m
    return (z - m) / jnp.sqrt(v + 1e-5) * g + b


def _tcz_body(aggA_ref, aggB_ref, xs_ref, wr_ref, wn_ref, b_ref, z_ref,
              st_ref):
    agg = jnp.concatenate([aggA_ref[0], aggA_ref[1], aggB_ref[0],
                           aggB_ref[1]], axis=1)
    z = (_dot1(agg, wr_ref[...]) + b_ref[...]) + _dot1(xs_ref[...],
                                                       wn_ref[...])
    z_ref[...] = z
    _sum_stats_body(z, st_ref, pl.program_id(0))


def _build_tcz(interpret=False):
    return pl.pallas_call(
        _tcz_body,
        grid=(NB,),
        in_specs=[
            pl.BlockSpec((2, BR, 32), lambda i: (0, i, 0)),
            pl.BlockSpec((2, BR, 32), lambda i: (0, i, 0)),
            pl.BlockSpec((BR, 128), lambda i: (i, 0)),
            pl.BlockSpec((128, 64), lambda i: (0, 0)),
            pl.BlockSpec((128, 64), lambda i: (0, 0)),
            pl.BlockSpec((1, 64), lambda i: (0, 0)),
        ],
        out_specs=[
            pl.BlockSpec((BR, 64), lambda i: (i, 0)),
            pl.BlockSpec((2, 64), lambda i: (0, 0)),
        ],
        out_shape=[
            jax.ShapeDtypeStruct((N, 64), F32),
            jax.ShapeDtypeStruct((2, 64), F32),
        ],
        interpret=interpret,
    )


def _tc3_body(z_ref, st_ref, g_ref, b_ref, w_ref, h_ref, zp_ref):
    h = jnp.maximum(_bn(z_ref[...], st_ref[...], g_ref[0], b_ref[0]), 0.0)
    h_ref[...] = h
    zp_ref[...] = _dot1(h, w_ref[...])


def _build_tc3(interpret=False):
    return pl.pallas_call(
        _tc3_body,
        grid=(NB,),
        in_specs=[
            pl.BlockSpec((BR, 64), lambda i: (i, 0)),
            pl.BlockSpec((2, 64), lambda i: (0, 0)),
            pl.BlockSpec((1, 64), lambda i: (0, 0)),
            pl.BlockSpec((1, 64), lambda i: (0, 0)),
            pl.BlockSpec((64, 128), lambda i: (0, 0)),
        ],
        out_specs=[
            pl.BlockSpec((BR, 64), lambda i: (i, 0)),
            pl.BlockSpec((BR, 128), lambda i: (i, 0)),
        ],
        out_shape=[
            jax.ShapeDtypeStruct((N, 64), F32),
            jax.ShapeDtypeStruct((N, 128), F32),
        ],
        interpret=interpret,
    )


def _tc4_body(agg_ref, zp_ref, wr_ref, b_ref, z_ref, st_ref):
    a = jnp.concatenate([agg_ref[0], agg_ref[1]], axis=1)
    z = (_dot1(a, wr_ref[...]) + b_ref[...]) + zp_ref[...]
    z_ref[...] = z
    _sum_stats_body(z, st_ref, pl.program_id(0))


def _build_tc4(interpret=False):
    return pl.pallas_call(
        _tc4_body,
        grid=(NB,),
        in_specs=[
            pl.BlockSpec((2, BR, 32), lambda i: (0, i, 0)),
            pl.BlockSpec((BR, 128), lambda i: (i, 0)),
            pl.BlockSpec((64, 128), lambda i: (0, 0)),
            pl.BlockSpec((1, 128), lambda i: (0, 0)),
        ],
        out_specs=[
            pl.BlockSpec((BR, 128), lambda i: (i, 0)),
            pl.BlockSpec((2, 128), lambda i: (0, 0)),
        ],
        out_shape=[
            jax.ShapeDtypeStruct((N, 128), F32),
            jax.ShapeDtypeStruct((2, 128), F32),
        ],
        interpret=interpret,
    )


def _tc5_body(z_ref, st_ref, g_ref, b_ref, h_ref):
    h_ref[...] = jnp.maximum(
        _bn(z_ref[...], st_ref[...], g_ref[0], b_ref[0]), 0.0)


def _build_tc5(interpret=False):
    return pl.pallas_call(
        _tc5_body,
        grid=(NB,),
        in_specs=[
            pl.BlockSpec((BR, 128), lambda i: (i, 0)),
            pl.BlockSpec((2, 128), lambda i: (0, 0)),
            pl.BlockSpec((1, 128), lambda i: (0, 0)),
            pl.BlockSpec((1, 128), lambda i: (0, 0)),
        ],
        out_specs=pl.BlockSpec((BR, 128), lambda i: (i, 0)),
        out_shape=jax.ShapeDtypeStruct((N, 128), F32),
        interpret=interpret,
    )


def _tc7_body(z_ref, st_ref, g_ref, b_ref, h1_ref, wc1_ref, bc1_ref,
              wc2_ref, bc2_ref, out_ref):
    bn = _bn(z_ref[...], st_ref[...], g_ref[0], b_ref[0])
    h3 = jnp.maximum(bn + h1_ref[...], 0.0)
    t = jnp.maximum(_dot1(h3, wc1_ref[...]) + bc1_ref[0], 0.0)
    out_ref[...] = _dot1(t, wc2_ref[...]) + bc2_ref[0]


def _build_tc7(interpret=False):
    return pl.pallas_call(
        _tc7_body,
        grid=(NB,),
        in_specs=[
            pl.BlockSpec((BR, 64), lambda i: (i, 0)),
            pl.BlockSpec((2, 64), lambda i: (0, 0)),
            pl.BlockSpec((1, 64), lambda i: (0, 0)),
            pl.BlockSpec((1, 64), lambda i: (0, 0)),
            pl.BlockSpec((BR, 64), lambda i: (i, 0)),
            pl.BlockSpec((64, 32), lambda i: (0, 0)),
            pl.BlockSpec((1, 32), lambda i: (0, 0)),
            pl.BlockSpec((32, 2), lambda i: (0, 0)),
            pl.BlockSpec((1, 2), lambda i: (0, 0)),
        ],
        out_specs=pl.BlockSpec((BR, 2), lambda i: (i, 0)),
        out_shape=jax.ShapeDtypeStruct((N, 2), F32),
        interpret=interpret,
    )


_tcz = _build_tcz()
_tc3 = _build_tc3()
_tc4 = _build_tc4()
_tc5 = _build_tc5()
_tc7 = _build_tc7()


def kernel(x, edge_index, edge_weight, W1r, b1r, W1n, g1, beta1, W2r, b2r,
           W2n, g2, beta2, W3r, b3r, W3n, g3, beta3, Wc1, bc1, Wc2, bc2):
    pad = E_PAD - E
    src = jnp.concatenate([edge_index[0], jnp.zeros((pad,), jnp.int32)])
    dst = jnp.concatenate(
        [edge_index[1], N + (jnp.arange(pad, dtype=jnp.int32) % 16)])
    ew = jnp.concatenate([edge_weight, jnp.zeros((pad,), F32)])

    x4 = x.reshape(4 * N, 32)
    aggA = _build_sc_agg(False, 4, 1, 0, 4 * N)(x4, src, dst, ew)
    aggB = _build_sc_agg(False, 4, 1, 2, 4 * N)(x4, src, dst, ew)
    z1, st1 = _tcz(aggA, aggB, x, W1r, W1n, b1r[None])
    h1, z2p = _tc3(z1, st1, g1[None], beta1[None], W2n)
    agg2 = _build_sc_agg(False, 2, 1, 0, 2 * N)(
        h1.reshape(2 * N, 32), src, dst, ew)
    z2, st2 = _tc4(agg2, z2p, W2r, b2r[None])
    h2 = _tc5(z2, st2, g2[None], beta2[None])
    h24 = h2.reshape(4 * N, 32)
    agg3A = _build_sc_agg(True, 4, 1, 0, 4 * N)(h24, src, dst, ew)
    agg3B = _build_sc_agg(True, 4, 1, 2, 4 * N)(h24, src, dst, ew)
    z3, st3 = _tcz(agg3A, agg3B, h2, W3r, W3n, b3r[None])
    return _tc7(z3, st3, g3[None], beta3[None], h1, Wc1, bc1[None], Wc2,
                bc2[None])


# speedup vs baseline: 1.0049x; 1.0049x over previous
import functools

import jax
import jax.numpy as jnp
from jax import lax
from jax.experimental import pallas as pl
from jax.experimental.pallas import tpu as pltpu
from jax.experimental.pallas import tpu_sc as plsc

N = 50000
E = 800000
BR = 2000
NB = N // BR
CHUNK = 128
N_TILES = 16
N_CHUNKS = 392
EDGES_PER_TILE = N_CHUNKS * CHUNK
E_PAD = N_TILES * EDGES_PER_TILE
ROWS_PER_TILE = 3200
DRAIN_ROWS = 400
N_ACC = N_TILES * ROWS_PER_TILE
F32 = jnp.float32


def _sc_agg_body(use_ew, qmul, cstride, poff, table, src, dst, ew, out,
                 src_i, dst_i, ew_b, msg, dbuf, acc, ssem, dsem, esem,
                 gsem):
    c = lax.axis_index("c")
    s = lax.axis_index("s")
    r0 = s * ROWS_PER_TILE

    def _zrow(i, _):
        dbuf[i, pl.ds(0, 16)] = jnp.zeros((16,), F32)
        dbuf[i, pl.ds(16, 16)] = jnp.zeros((16,), F32)
        return 0
    lax.fori_loop(0, DRAIN_ROWS, _zrow, 0)
    for k in range(ROWS_PER_TILE // DRAIN_ROWS):
        pltpu.sync_copy(dbuf, acc.at[pl.ds(r0 + k * DRAIN_ROWS, DRAIN_ROWS)])
    plsc.subcore_barrier()

    base = s * EDGES_PER_TILE
    coff = c * cstride + poff

    def start_idx(j, b):
        off = base + j * CHUNK
        pltpu.async_copy(src.at[pl.ds(off, CHUNK)], src_i.at[b], ssem.at[b])
        pltpu.async_copy(dst.at[pl.ds(off, CHUNK)], dst_i.at[b], dsem.at[b])
        if use_ew:
            pltpu.async_copy(ew.at[pl.ds(off, CHUNK)], ew_b.at[b],
                             esem.at[b])

    def gather_chunk(b):
        pltpu.make_async_copy(src.at[pl.ds(base, CHUNK)], src_i.at[b],
                              ssem.at[b]).wait()
        for f in range(CHUNK // 16):
            sl = pl.ds(f * 16, 16)
            if qmul == 1:
                src_i[b, sl] = src_i[b, sl] + coff
            else:
                src_i[b, sl] = src_i[b, sl] * qmul + coff
        pltpu.async_copy(table.at[src_i.at[b]], msg.at[b], gsem.at[b])

    def wait_gather(b):
        pltpu.make_async_copy(table.at[src_i.at[b]], msg.at[b],
                              gsem.at[b]).wait()

    def do_scatter(b):
        if use_ew:
            pltpu.make_async_copy(ew.at[pl.ds(base, CHUNK)], ew_b.at[b],
                                  esem.at[b]).wait()
            for g in range(CHUNK // 16):
                w16 = ew_b[b, pl.ds(g * 16, 16)]
                for el in range(16):
                    e = g * 16 + el
                    wv = jnp.full((16,), w16[el], F32)
                    msg[b, e, pl.ds(0, 16)] = msg[b, e, pl.ds(0, 16)] * wv
                    msg[b, e, pl.ds(16, 16)] = msg[b, e, pl.ds(16, 16)] * wv
        pltpu.make_async_copy(dst.at[pl.ds(base, CHUNK)], dst_i.at[b],
                              dsem.at[b]).wait()
        pltpu.sync_copy(msg.at[b], acc.at[dst_i.at[b]], add=True)

    start_idx(0, 0)
    start_idx(1, 1)
    gather_chunk(0)

    def _outer(t, _):
        j0 = 2 * t
        for b in range(2):
            wait_gather(b)
            gather_chunk(1 - b)
            do_scatter(b)
            start_idx(j0 + b + 2, b)
        return 0
    lax.fori_loop(0, (N_CHUNKS - 2) // 2, _outer, 0)
    wait_gather(0)
    gather_chunk(1)
    do_scatter(0)
    wait_gather(1)
    do_scatter(1)
    plsc.subcore_barrier()

    for k in range(ROWS_PER_TILE // DRAIN_ROWS):
        r = r0 + k * DRAIN_ROWS
        pltpu.sync_copy(acc.at[pl.ds(r, DRAIN_ROWS)], dbuf)
        pltpu.sync_copy(dbuf, out.at[c, pl.ds(r, DRAIN_ROWS)])


@functools.cache
def _build_sc_agg(use_ew, qmul, cstride, poff, table_rows):
    mesh = plsc.VectorSubcoreMesh(core_axis_name="c", subcore_axis_name="s",
                                  num_cores=2, num_subcores=N_TILES)
    return pl.kernel(
        functools.partial(_sc_agg_body, use_ew, qmul, cstride, poff),
        out_type=jax.ShapeDtypeStruct((2, N_ACC, 32), F32),
        mesh=mesh,
        scratch_types=[
            pltpu.VMEM((2, CHUNK), jnp.int32),
            pltpu.VMEM((2, CHUNK), jnp.int32),
            pltpu.VMEM((2, CHUNK), F32),
            pltpu.VMEM((2, CHUNK, 32), F32),
            pltpu.VMEM((DRAIN_ROWS, 32), F32),
            pltpu.VMEM_SHARED((N_ACC, 32), F32),
            pltpu.SemaphoreType.DMA((2,)),
            pltpu.SemaphoreType.DMA((2,)),
            pltpu.SemaphoreType.DMA((2,)),
            pltpu.SemaphoreType.DMA((2,)),
        ],
        compiler_params=pltpu.CompilerParams(use_tc_tiling_on_sc=False),
    )


BF16 = jnp.bfloat16


def _dot1(a, b):
    return jnp.dot(a.astype(BF16), b.astype(BF16), preferred_element_type=F32)


def _sum_stats_body(z, st_ref, i):
    @pl.when(i == 0)
    def _():
        st_ref[...] = jnp.zeros_like(st_ref)
    st_ref[...] += jnp.stack([jnp.sum(z, 0), jnp.sum(z * z, 0)])


def _bn(z, st, g, b):
    m = st[0] / N
    v = st[1] / N - m * m
    return (z - m) / jnp.sqrt(v + 1e-5) * g + b


def _tcz_body(aggA_ref, aggB_ref, xs_ref, wr_ref, wn_ref, b_ref, z_ref,
              st_ref):
    agg = jnp.concatenate([aggA_ref[0], aggA_ref[1], aggB_ref[0],
                           aggB_ref[1]], axis=1)
    z = (_dot1(agg, wr_ref[...]) + b_ref[...]) + _dot1(xs_ref[...],
                                                       wn_ref[...])
    z_ref[...] = z
    _sum_stats_body(z, st_ref, pl.program_id(0))


def _build_tcz(interpret=False):
    return pl.pallas_call(
        _tcz_body,
        grid=(NB,),
        in_specs=[
            pl.BlockSpec((2, BR, 32), lambda i: (0, i, 0)),
            pl.BlockSpec((2, BR, 32), lambda i: (0, i, 0)),
            pl.BlockSpec((BR, 128), lambda i: (i, 0)),
            pl.BlockSpec((128, 64), lambda i: (0, 0)),
            pl.BlockSpec((128, 64), lambda i: (0, 0)),
            pl.BlockSpec((1, 64), lambda i: (0, 0)),
        ],
        out_specs=[
            pl.BlockSpec((BR, 64), lambda i: (i, 0)),
            pl.BlockSpec((2, 64), lambda i: (0, 0)),
        ],
        out_shape=[
            jax.ShapeDtypeStruct((N, 64), F32),
            jax.ShapeDtypeStruct((2, 64), F32),
        ],
        interpret=interpret,
    )


def _tc3_body(z_ref, st_ref, g_ref, b_ref, w_ref, h_ref, zp_ref):
    h = jnp.maximum(_bn(z_ref[...], st_ref[...], g_ref[0], b_ref[0]), 0.0)
    h_ref[...] = h
    zp_ref[...] = _dot1(h, w_ref[...])


def _build_tc3(interpret=False):
    return pl.pallas_call(
        _tc3_body,
        grid=(NB,),
        in_specs=[
            pl.BlockSpec((BR, 64), lambda i: (i, 0)),
            pl.BlockSpec((2, 64), lambda i: (0, 0)),
            pl.BlockSpec((1, 64), lambda i: (0, 0)),
            pl.BlockSpec((1, 64), lambda i: (0, 0)),
            pl.BlockSpec((64, 128), lambda i: (0, 0)),
        ],
        out_specs=[
            pl.BlockSpec((BR, 64), lambda i: (i, 0)),
            pl.BlockSpec((BR, 128), lambda i: (i, 0)),
        ],
        out_shape=[
            jax.ShapeDtypeStruct((N, 64), F32),
            jax.ShapeDtypeStruct((N, 128), F32),
        ],
        interpret=interpret,
    )


def _tc4_body(agg_ref, zp_ref, wr_ref, b_ref, z_ref, st_ref):
    a = jnp.concatenate([agg_ref[0], agg_ref[1]], axis=1)
    z = (_dot1(a, wr_ref[...]) + b_ref[...]) + zp_ref[...]
    z_ref[...] = z
    _sum_stats_body(z, st_ref, pl.program_id(0))


def _build_tc4(interpret=False):
    return pl.pallas_call(
        _tc4_body,
        grid=(NB,),
        in_specs=[
            pl.BlockSpec((2, BR, 32), lambda i: (0, i, 0)),
            pl.BlockSpec((BR, 128), lambda i: (i, 0)),
            pl.BlockSpec((64, 128), lambda i: (0, 0)),
            pl.BlockSpec((1, 128), lambda i: (0, 0)),
        ],
        out_specs=[
            pl.BlockSpec((BR, 128), lambda i: (i, 0)),
            pl.BlockSpec((2, 128), lambda i: (0, 0)),
        ],
        out_shape=[
            jax.ShapeDtypeStruct((N, 128), F32),
            jax.ShapeDtypeStruct((2, 128), F32),
        ],
        interpret=interpret,
    )


def _tc5_body(z_ref, st_ref, g_ref, b_ref, h_ref):
    h_ref[...] = jnp.maximum(
        _bn(z_ref[...], st_ref[...], g_ref[0], b_ref[0]), 0.0)


def _build_tc5(interpret=False):
    return pl.pallas_call(
        _tc5_body,
        grid=(NB,),
        in_specs=[
            pl.BlockSpec((BR, 128), lambda i: (i, 0)),
            pl.BlockSpec((2, 128), lambda i: (0, 0)),
            pl.BlockSpec((1, 128), lambda i: (0, 0)),
            pl.BlockSpec((1, 128), lambda i: (0, 0)),
        ],
        out_specs=pl.BlockSpec((BR, 128), lambda i: (i, 0)),
        out_shape=jax.ShapeDtypeStruct((N, 128), F32),
        interpret=interpret,
    )


def _tc7_body(z_ref, st_ref, g_ref, b_ref, h1_ref, wc1_ref, bc1_ref,
              wc2_ref, bc2_ref, out_ref):
    bn = _bn(z_ref[...], st_ref[...], g_ref[0], b_ref[0])
    h3 = jnp.maximum(bn + h1_ref[...], 0.0)
    t = jnp.maximum(_dot1(h3, wc1_ref[...]) + bc1_ref[0], 0.0)
    out_ref[...] = _dot1(t, wc2_ref[...]) + bc2_ref[0]


def _build_tc7(interpret=False):
    return pl.pallas_call(
        _tc7_body,
        grid=(NB,),
        in_specs=[
            pl.BlockSpec((BR, 64), lambda i: (i, 0)),
            pl.BlockSpec((2, 64), lambda i: (0, 0)),
            pl.BlockSpec((1, 64), lambda i: (0, 0)),
            pl.BlockSpec((1, 64), lambda i: (0, 0)),
            pl.BlockSpec((BR, 64), lambda i: (i, 0)),
            pl.BlockSpec((64, 32), lambda i: (0, 0)),
            pl.BlockSpec((1, 32), lambda i: (0, 0)),
            pl.BlockSpec((32, 2), lambda i: (0, 0)),
            pl.BlockSpec((1, 2), lambda i: (0, 0)),
        ],
        out_specs=pl.BlockSpec((BR, 2), lambda i: (i, 0)),
        out_shape=jax.ShapeDtypeStruct((N, 2), F32),
        interpret=interpret,
    )


_tcz = _build_tcz()
_tc3 = _build_tc3()
_tc4 = _build_tc4()
_tc5 = _build_tc5()
_tc7 = _build_tc7()


def kernel(x, edge_index, edge_weight, W1r, b1r, W1n, g1, beta1, W2r, b2r,
           W2n, g2, beta2, W3r, b3r, W3n, g3, beta3, Wc1, bc1, Wc2, bc2):
    pad = E_PAD - E
    src = jnp.concatenate([edge_index[0], jnp.zeros((pad,), jnp.int32)])
    dst = jnp.concatenate(
        [edge_index[1], N + (jnp.arange(pad, dtype=jnp.int32) % 16)])
    ew = jnp.concatenate([edge_weight, jnp.zeros((pad,), F32)])

    x4 = x.reshape(4 * N, 32)
    aggA = _build_sc_agg(False, 4, 1, 0, 4 * N)(x4, src, dst, ew)
    aggB = _build_sc_agg(False, 4, 1, 2, 4 * N)(x4, src, dst, ew)
    z1, st1 = _tcz(aggA, aggB, x, W1r, W1n, b1r[None])
    h1, z2p = _tc3(z1, st1, g1[None], beta1[None], W2n)
    agg2 = _build_sc_agg(False, 2, 1, 0, 2 * N)(
        h1.reshape(2 * N, 32), src, dst, ew)
    z2, st2 = _tc4(agg2, z2p, W2r, b2r[None])
    h2 = _tc5(z2, st2, g2[None], beta2[None])
    h24 = h2.reshape(4 * N, 32)
    agg3A = _build_sc_agg(True, 4, 1, 0, 4 * N)(h24, src, dst, ew)
    agg3B = _build_sc_agg(True, 4, 1, 2, 4 * N)(h24, src, dst, ew)
    z3, st3 = _tcz(agg3A, agg3B, h2, W3r, W3n, b3r[None])
    return _tc7(z3, st3, g3[None], beta3[None], h1, Wc1, bc1[None], Wc2,
                bc2[None])
